# baseline (speedup 1.0000x reference)
import functools

import jax
import jax.numpy as jnp
from jax import lax
from jax.experimental import pallas as pl
from jax.experimental.pallas import tpu as pltpu
from jax.experimental.pallas import tpu_sc as plsc

VOCAB = 100000
EMBED = 128
B, L = 4096, 200

NUM_CORES = 2
NUM_SUBCORES = 16
NW = NUM_CORES * NUM_SUBCORES
TOTAL = B * L
PER_W = TOTAL // NW
CHUNK = 128
N_CHUNK = PER_W // CHUNK


NBUF = 6
AHEAD = 4


def _make_gather():
    mesh = plsc.VectorSubcoreMesh(core_axis_name="c", subcore_axis_name="s")

    @functools.partial(
        pl.kernel,
        mesh=mesh,
        out_type=jax.ShapeDtypeStruct((TOTAL, EMBED), jnp.float32),
        scratch_types=[
            pltpu.VMEM((N_CHUNK, CHUNK), jnp.int32),
            pltpu.VMEM((NBUF, CHUNK, EMBED), jnp.float32),
        ] + [pltpu.SemaphoreType.DMA] * (2 * NBUF),
    )
    def gather_kernel(table_hbm, idx_hbm, out_hbm, idx_v, rows_v, *sems):
        gsem = sems[:NBUF]
        osem = sems[NBUF:]
        wid = lax.axis_index("s") * NUM_CORES + lax.axis_index("c")
        pltpu.sync_copy(idx_hbm.at[pl.ds(wid * N_CHUNK, N_CHUNK)], idx_v)
        row_base = wid * PER_W

        def g_start(c, b):
            pltpu.async_copy(table_hbm.at[idx_v.at[c]], rows_v.at[b], gsem[b])

        def g_wait(b):
            pltpu.make_async_copy(table_hbm.at[pl.ds(0, CHUNK)],
                                  rows_v.at[b], gsem[b]).wait()

        def o_start(c, b):
            pltpu.async_copy(rows_v.at[b],
                             out_hbm.at[pl.ds(row_base + c * CHUNK, CHUNK)],
                             osem[b])

        def o_wait(b):
            pltpu.make_async_copy(rows_v.at[b], out_hbm.at[pl.ds(0, CHUNK)],
                                  osem[b]).wait()

        def visit(c, b, do_owait, do_gstart):
            g_wait(b)
            o_start(c, b)
            if do_gstart:
                if do_owait:
                    o_wait((b + AHEAD) % NBUF)
                g_start(c + AHEAD, (b + AHEAD) % NBUF)

        for c in range(AHEAD):
            g_start(c, c)
        for c in range(NBUF - AHEAD):
            visit(c, c, do_owait=False, do_gstart=True)

        start = NBUF - AHEAD
        n_steady = (N_CHUNK - AHEAD - start) // NBUF

        def body(t, carry):
            for k in range(NBUF):
                b = (start + k) % NBUF
                visit(NBUF * t + start + k, b, do_owait=True, do_gstart=True)
            return carry

        lax.fori_loop(0, n_steady, body, 0)

        for c in range(start + n_steady * NBUF, N_CHUNK):
            visit(c, c % NBUF, do_owait=True, do_gstart=(c + AHEAD < N_CHUNK))
        for c in range(N_CHUNK - NBUF, N_CHUNK):
            o_wait(c % NBUF)

    return gather_kernel


_gather = _make_gather()


def kernel(input_ids, token_type_ids, word_embeddings, position_embeddings,
           token_type_embeddings):
    idx = input_ids.reshape(NW * N_CHUNK, CHUNK).astype(jnp.int32)
    out = _gather(word_embeddings, idx)
    return out.reshape(B, L, EMBED)


# speedup vs baseline: 1.0015x; 1.0015x over previous
import functools

import jax
import jax.numpy as jnp
from jax import lax
from jax.experimental import pallas as pl
from jax.experimental.pallas import tpu as pltpu
from jax.experimental.pallas import tpu_sc as plsc

VOCAB = 100000
EMBED = 128
B, L = 4096, 200

NUM_CORES = 2
NUM_SUBCORES = 16
NW = NUM_CORES * NUM_SUBCORES
TOTAL = B * L
PER_W = TOTAL // NW
CHUNK = 128
N_CHUNK = PER_W // CHUNK

NBUF = 6
AHEAD = 3


def _make_gather():
    mesh = plsc.VectorSubcoreMesh(core_axis_name="c", subcore_axis_name="s")

    @functools.partial(
        pl.kernel,
        mesh=mesh,
        out_type=jax.ShapeDtypeStruct((TOTAL, EMBED), jnp.float32),
        scratch_types=[
            pltpu.VMEM((N_CHUNK, CHUNK), jnp.int32),
            pltpu.VMEM((NBUF, CHUNK, EMBED), jnp.float32),
        ] + [pltpu.SemaphoreType.DMA] * (2 * NBUF),
    )
    def gather_kernel(table_hbm, idx_hbm, out_hbm, idx_v, rows_v, *sems):
        gsem = sems[:NBUF]
        osem = sems[NBUF:]
        wid = lax.axis_index("s") * NUM_CORES + lax.axis_index("c")
        pltpu.sync_copy(idx_hbm.at[pl.ds(wid * N_CHUNK, N_CHUNK)], idx_v)
        row_base = wid * PER_W

        def g_start(c, b):
            pltpu.async_copy(table_hbm.at[idx_v.at[c]], rows_v.at[b], gsem[b])

        def g_wait(b):
            pltpu.make_async_copy(table_hbm.at[pl.ds(0, CHUNK)],
                                  rows_v.at[b], gsem[b]).wait()

        def o_start(c, b):
            pltpu.async_copy(rows_v.at[b],
                             out_hbm.at[pl.ds(row_base + c * CHUNK, CHUNK)],
                             osem[b])

        def o_wait(b):
            pltpu.make_async_copy(rows_v.at[b], out_hbm.at[pl.ds(0, CHUNK)],
                                  osem[b]).wait()

        def visit(c, b, do_owait, do_gstart):
            g_wait(b)
            o_start(c, b)
            if do_gstart:
                if do_owait:
                    o_wait((b + AHEAD) % NBUF)
                g_start(c + AHEAD, (b + AHEAD) % NBUF)

        for c in range(AHEAD):
            g_start(c, c)
        for c in range(NBUF - AHEAD):
            visit(c, c, do_owait=False, do_gstart=True)

        start = NBUF - AHEAD
        n_steady = (N_CHUNK - AHEAD - start) // NBUF

        def body(t, carry):
            for k in range(NBUF):
                b = (start + k) % NBUF
                visit(NBUF * t + start + k, b, do_owait=True, do_gstart=True)
            return carry

        lax.fori_loop(0, n_steady, body, 0)

        for c in range(start + n_steady * NBUF, N_CHUNK):
            visit(c, c % NBUF, do_owait=True, do_gstart=(c + AHEAD < N_CHUNK))
        for c in range(N_CHUNK - NBUF, N_CHUNK):
            o_wait(c % NBUF)

    return gather_kernel


_gather = _make_gather()


def kernel(input_ids, token_type_ids, word_embeddings, position_embeddings,
           token_type_embeddings):
    idx = input_ids.reshape(NW * N_CHUNK, CHUNK).astype(jnp.int32)
    out = _gather(word_embeddings, idx)
    return out.reshape(B, L, EMBED)
